# baseline (speedup 1.0000x reference)
import jax
import jax.numpy as jnp
from jax.experimental import pallas as pl
from jax.experimental.pallas import tpu as pltpu

_BPG = 4


def kernel(x, row_emb, col_emb):
    b = x.shape[0]
    h, w = x.shape[-2], x.shape[-1]
    d = row_emb.shape[1]

    def body(col_ref, row_ref, out_ref):
        col = col_ref[:w, :]
        row = row_ref[:h, :]
        out_ref[:, :, :, 0:d] = jnp.broadcast_to(
            col[None, None, :, :], (_BPG, h, w, d))
        out_ref[:, :, :, d:2 * d] = jnp.broadcast_to(
            row[None, :, None, :], (_BPG, h, w, d))

    out = pl.pallas_call(
        body,
        grid=(b // _BPG,),
        in_specs=[
            pl.BlockSpec(col_emb.shape, lambda i: (0, 0)),
            pl.BlockSpec(row_emb.shape, lambda i: (0, 0)),
        ],
        out_specs=pl.BlockSpec((_BPG, h, w, 2 * d), lambda i: (i, 0, 0, 0)),
        out_shape=jax.ShapeDtypeStruct((b, h, w, 2 * d), jnp.float32),
    )(col_emb, row_emb)
    return jnp.transpose(out, (0, 3, 1, 2))
